# baseline (speedup 1.0000x reference)
import functools

import jax
import jax.numpy as jnp
import numpy as np
from jax import lax
from jax.experimental import pallas as pl
from jax.experimental.pallas import tpu as pltpu
from jax.experimental.pallas import tpu_sc as plsc

_CUTOFF = 5.0
_N = 10000
_E = 320000
_D = 128
_F = 64

_NC = 2
_NS = 16
_NW = _NC * _NS
_L = 16
_CH = 128
_NCHUNKS = _E // _CH
_RPT = 640
_NPAD = _RPT * _NS


def _filter_body(d_ref, r_ref, c_ref, p_ref, o_ref):
    d = d_ref[:, 0, :]
    scaled = d * (2.0 / _CUTOFF) - 1.0
    acc = jnp.zeros(d.shape, jnp.float32)
    for j in range(_F):
        w1j = p_ref[0, j]
        b1j = p_ref[1, j]
        w2j = p_ref[2, j]
        acc = acc + w2j * jnp.tanh(scaled * w1j + b1j)
    acc = acc + p_ref[3, 0]
    cut = jnp.where(d <= _CUTOFF,
                    0.5 * (jnp.cos(d * (np.pi / _CUTOFF)) + 1.0),
                    0.0)
    o_ref[:, 0, :] = r_ref[:, 0, :]
    o_ref[:, 1, :] = c_ref[:, 0, :]
    o_ref[:, 2, :] = lax.bitcast_convert_type(acc * cut, jnp.int32)


_BR_S = 250

_filter_call = pl.pallas_call(
    _filter_body,
    grid=(_NCHUNKS // _BR_S,),
    in_specs=[
        pl.BlockSpec((_BR_S, 1, _CH), lambda i: (i, 0, 0)),
        pl.BlockSpec((_BR_S, 1, _CH), lambda i: (i, 0, 0)),
        pl.BlockSpec((_BR_S, 1, _CH), lambda i: (i, 0, 0)),
        pl.BlockSpec((4, _F), lambda i: (0, 0)),
    ],
    out_specs=pl.BlockSpec((_BR_S, 3, _CH), lambda i: (i, 0, 0)),
    out_shape=jax.ShapeDtypeStruct((_NCHUNKS, 3, _CH), jnp.int32),
)


_sc_mesh = plsc.VectorSubcoreMesh(core_axis_name="c", subcore_axis_name="s",
                                  num_cores=_NC)


_MAXLOC = (_NCHUNKS + _NW - 1) // _NW


@functools.partial(
    pl.kernel,
    mesh=_sc_mesh,
    out_type=jax.ShapeDtypeStruct((_NC * _N, _D), jnp.float32),
    scratch_types=[
        pltpu.VMEM((3, _CH), jnp.int32),
        pltpu.VMEM((_CH, _D), jnp.float32),
        pltpu.VMEM((3, _CH), jnp.int32),
        pltpu.VMEM((_CH, _D), jnp.float32),
        pltpu.VMEM_SHARED((_NPAD, _D), jnp.float32),
        pltpu.SemaphoreType.DMA,
        pltpu.SemaphoreType.DMA,
    ],
)
def _sc_scatter(x_hbm, packed_hbm, out_hbm,
                idx0, rows0, idx1, rows1, acc, sem0, sem1):
    cid = lax.axis_index("c")
    sid = lax.axis_index("s")
    wid = sid * _NC + cid
    bufs = ((idx0, rows0, sem0), (idx1, rows1, sem1))

    def _zrow(e, carry):
        for j in range(_D // _L):
            rows0[e, pl.ds(j * _L, _L)] = jnp.zeros((_L,), jnp.float32)
        return carry
    lax.fori_loop(0, _CH, _zrow, 0)
    for z in range(_RPT // _CH):
        pltpu.sync_copy(rows0.at[pl.ds(0, _CH)],
                        acc.at[pl.ds(sid * _RPT + z * _CH, _CH)])

    nloc = (_NCHUNKS - wid + (_NW - 1)) // _NW

    def _fetch_idx(i, b):
        pltpu.sync_copy(packed_hbm.at[wid + i * _NW], bufs[b][0])

    def _start_gather(b):
        pltpu.async_copy(x_hbm.at[bufs[b][0].at[1]], bufs[b][1], bufs[b][2])

    def _wait_gather(b):
        pltpu.make_async_copy(x_hbm.at[bufs[b][0].at[1]], bufs[b][1],
                              bufs[b][2]).wait()

    def _scale_rows(b):
        idx, rows, sem = bufs[b]

        def _scale(g, c2):
            svec = lax.bitcast_convert_type(idx[2, pl.ds(g * _L, _L)],
                                            jnp.float32)
            for k in range(_L):
                e = g * _L + k
                fv = jnp.full((_L,), svec[k], jnp.float32)
                for j in range(_D // _L):
                    rows[e, pl.ds(j * _L, _L)] = rows[e, pl.ds(j * _L, _L)] * fv
            return c2
        lax.fori_loop(0, _CH // _L, _scale, 0)

    def _scatter(b):
        pltpu.sync_copy(bufs[b][1], acc.at[bufs[b][0].at[0]], add=True)

    _fetch_idx(0, 0)
    _start_gather(0)
    _wait_gather(0)
    plsc.subcore_barrier()

    def _step(i2, carry):
        for b in range(2):
            i = i2 * 2 + b

            @pl.when(i < nloc)
            def _proc():
                @pl.when(i + 1 < nloc)
                def _pref():
                    _fetch_idx(i + 1, 1 - b)
                    _start_gather(1 - b)

                _scale_rows(b)

                @pl.when(i + 1 < nloc)
                def _drain():
                    _wait_gather(1 - b)

                _scatter(b)
        return carry
    lax.fori_loop(0, (_MAXLOC + 1) // 2, _step, 0)

    plsc.subcore_barrier()
    last = _N - (_NS - 1) * _RPT

    @pl.when(sid < _NS - 1)
    def _copy_main():
        pltpu.sync_copy(acc.at[pl.ds(sid * _RPT, _RPT)],
                        out_hbm.at[pl.ds(cid * _N + sid * _RPT, _RPT)])

    @pl.when(sid == _NS - 1)
    def _copy_last():
        pltpu.sync_copy(acc.at[pl.ds((_NS - 1) * _RPT, last)],
                        out_hbm.at[pl.ds(cid * _N + (_NS - 1) * _RPT, last)])


def _inter_body(p_ref, w1_ref, b1_ref, w2_ref, b2_ref, o_ref):
    z = p_ref[0] + p_ref[1]
    h = jnp.dot(z, w1_ref[...], preferred_element_type=jnp.float32,
                precision=jax.lax.Precision.HIGHEST)
    h = h + b1_ref[...]
    h = jnp.maximum(h, 0.0) + jnp.log1p(jnp.exp(-jnp.abs(h)))
    o = jnp.dot(h, w2_ref[...], preferred_element_type=jnp.float32,
                precision=jax.lax.Precision.HIGHEST)
    o_ref[...] = o + b2_ref[...]


_BR_I = 1000

_inter_call = pl.pallas_call(
    _inter_body,
    grid=(_N // _BR_I,),
    in_specs=[
        pl.BlockSpec((_NC, _BR_I, _D), lambda i: (0, i, 0)),
        pl.BlockSpec((_D, _D), lambda i: (0, 0)),
        pl.BlockSpec((1, _D), lambda i: (0, 0)),
        pl.BlockSpec((_D, _D), lambda i: (0, 0)),
        pl.BlockSpec((1, _D), lambda i: (0, 0)),
    ],
    out_specs=pl.BlockSpec((_BR_I, _D), lambda i: (i, 0)),
    out_shape=jax.ShapeDtypeStruct((_N, _D), jnp.float32),
)


def kernel(x, edge_index, distances, edge_attr,
           W1f, b1f, W2f, b2f, W1i, b1i, W2i, b2i):
    del edge_attr
    row = edge_index[0].astype(jnp.int32)
    col = edge_index[1].astype(jnp.int32)
    params = jnp.stack([
        W1f[0],
        b1f,
        jnp.sum(W2f, axis=1),
        jnp.full((_F,), jnp.sum(b2f), jnp.float32),
    ])
    packed = _filter_call(distances.reshape(_NCHUNKS, 1, _CH),
                          row.reshape(_NCHUNKS, 1, _CH),
                          col.reshape(_NCHUNKS, 1, _CH),
                          params)
    partials = _sc_scatter(x, packed)
    out = _inter_call(partials.reshape(_NC, _N, _D),
                      W1i, b1i.reshape(1, _D), W2i, b2i.reshape(1, _D))
    return out


# speedup vs baseline: 1.7835x; 1.7835x over previous
import functools

import jax
import jax.numpy as jnp
import numpy as np
from jax import lax
from jax.experimental import pallas as pl
from jax.experimental.pallas import tpu as pltpu
from jax.experimental.pallas import tpu_sc as plsc

_CUTOFF = 5.0
_N = 10000
_E = 320000
_D = 128
_F = 64

_NC = 2
_NS = 16
_NW = _NC * _NS
_L = 16
_CH = 128
_NCHUNKS = _E // _CH
_RPT = 640
_NPAD = _RPT * _NS


def _filter_body(d_ref, p_ref, s_ref):
    d = d_ref[...]
    scaled = d * (2.0 / _CUTOFF) - 1.0
    acc = jnp.zeros(d.shape, jnp.float32)
    for j in range(_F):
        w1j = p_ref[0, j]
        b1j = p_ref[1, j]
        w2j = p_ref[2, j]
        acc = acc + w2j * jnp.tanh(scaled * w1j + b1j)
    acc = acc + p_ref[3, 0]
    cut = jnp.where(d <= _CUTOFF,
                    0.5 * (jnp.cos(d * (np.pi / _CUTOFF)) + 1.0),
                    0.0)
    s_ref[...] = acc * cut


_ROWS_S = 320
_COLS_S = 1000
_BR_S = 8

_filter_call = pl.pallas_call(
    _filter_body,
    grid=(_ROWS_S // _BR_S,),
    in_specs=[
        pl.BlockSpec((_BR_S, _COLS_S), lambda i: (i, 0)),
        pl.BlockSpec((4, _F), lambda i: (0, 0)),
    ],
    out_specs=pl.BlockSpec((_BR_S, _COLS_S), lambda i: (i, 0)),
    out_shape=jax.ShapeDtypeStruct((_ROWS_S, _COLS_S), jnp.float32),
)


_sc_mesh = plsc.VectorSubcoreMesh(core_axis_name="c", subcore_axis_name="s",
                                  num_cores=_NC)


_MAXLOC = (_NCHUNKS + _NW - 1) // _NW


@functools.partial(
    pl.kernel,
    mesh=_sc_mesh,
    out_type=jax.ShapeDtypeStruct((_NC * _N, _D), jnp.float32),
    scratch_types=[
        pltpu.VMEM((3, _CH), jnp.int32),
        pltpu.VMEM((_CH, _D), jnp.float32),
        pltpu.VMEM((3, _CH), jnp.int32),
        pltpu.VMEM((_CH, _D), jnp.float32),
        pltpu.VMEM((_CH,), jnp.int32),
        pltpu.VMEM((_CH,), jnp.int32),
        pltpu.VMEM_SHARED((_NPAD, _D), jnp.float32),
        pltpu.SemaphoreType.DMA,
        pltpu.SemaphoreType.DMA,
        pltpu.SemaphoreType.DMA,
        pltpu.SemaphoreType.DMA,
    ],
)
def _sc_scatter(x_hbm, packed_hbm, out_hbm,
                idx0, rows0, idx1, rows1, rowv0, rowv1,
                acc, sem0, sem1, isem0, isem1):
    cid = lax.axis_index("c")
    sid = lax.axis_index("s")
    wid = sid * _NC + cid
    bufs = ((idx0, rows0, sem0, isem0, rowv0),
            (idx1, rows1, sem1, isem1, rowv1))

    def _zrow(e, carry):
        for j in range(_D // _L):
            rows0[e, pl.ds(j * _L, _L)] = jnp.zeros((_L,), jnp.float32)
        return carry
    lax.fori_loop(0, _CH, _zrow, 0)
    for z in range(_RPT // _CH):
        pltpu.sync_copy(rows0.at[pl.ds(0, _CH)],
                        acc.at[pl.ds(sid * _RPT + z * _CH, _CH)])

    nloc = (_NCHUNKS - wid + (_NW - 1)) // _NW

    def _start_idx(i, b):
        pltpu.async_copy(packed_hbm.at[wid + i * _NW], bufs[b][0], bufs[b][3])

    def _wait_idx(i, b):
        pltpu.make_async_copy(packed_hbm.at[wid + i * _NW], bufs[b][0],
                              bufs[b][3]).wait()

    def _start_gather(b):
        pltpu.async_copy(x_hbm.at[bufs[b][0].at[1]], bufs[b][1], bufs[b][2])

    def _wait_gather(b):
        pltpu.make_async_copy(x_hbm.at[bufs[b][0].at[1]], bufs[b][1],
                              bufs[b][2]).wait()

    def _scale_rows(b):
        idx, rows, sem, isem, rowv = bufs[b]

        def _scale(g, c2):
            rowv[pl.ds(g * _L, _L)] = idx[0, pl.ds(g * _L, _L)]
            svec = lax.bitcast_convert_type(idx[2, pl.ds(g * _L, _L)],
                                            jnp.float32)
            for k in range(_L):
                e = g * _L + k
                fv = jnp.full((_L,), svec[k], jnp.float32)
                for j in range(_D // _L):
                    rows[e, pl.ds(j * _L, _L)] = rows[e, pl.ds(j * _L, _L)] * fv
            return c2
        lax.fori_loop(0, _CH // _L, _scale, 0)

    def _scatter(b):
        pltpu.sync_copy(bufs[b][1], acc.at[bufs[b][4]], add=True)

    _start_idx(0, 0)
    _wait_idx(0, 0)
    _start_gather(0)
    _wait_gather(0)

    @pl.when(1 < nloc)
    def _pre1():
        _start_idx(1, 1)
    plsc.subcore_barrier()

    def _step(i2, carry):
        for b in range(2):
            i = i2 * 2 + b

            @pl.when(i < nloc)
            def _proc():
                @pl.when(i + 1 < nloc)
                def _pref():
                    _wait_idx(i + 1, 1 - b)
                    _start_gather(1 - b)

                _scale_rows(b)

                @pl.when(i + 1 < nloc)
                def _drain():
                    _wait_gather(1 - b)

                @pl.when(i + 2 < nloc)
                def _pref2():
                    _start_idx(i + 2, b)

                _scatter(b)
        return carry
    lax.fori_loop(0, (_MAXLOC + 1) // 2, _step, 0)

    plsc.subcore_barrier()
    last = _N - (_NS - 1) * _RPT

    @pl.when(sid < _NS - 1)
    def _copy_main():
        pltpu.sync_copy(acc.at[pl.ds(sid * _RPT, _RPT)],
                        out_hbm.at[pl.ds(cid * _N + sid * _RPT, _RPT)])

    @pl.when(sid == _NS - 1)
    def _copy_last():
        pltpu.sync_copy(acc.at[pl.ds((_NS - 1) * _RPT, last)],
                        out_hbm.at[pl.ds(cid * _N + (_NS - 1) * _RPT, last)])


def _inter_body(p_ref, w1_ref, b1_ref, w2_ref, b2_ref, o_ref):
    z = p_ref[0] + p_ref[1]
    h = jnp.dot(z, w1_ref[...], preferred_element_type=jnp.float32,
                precision=jax.lax.Precision.HIGHEST)
    h = h + b1_ref[...]
    h = jnp.maximum(h, 0.0) + jnp.log1p(jnp.exp(-jnp.abs(h)))
    o = jnp.dot(h, w2_ref[...], preferred_element_type=jnp.float32,
                precision=jax.lax.Precision.HIGHEST)
    o_ref[...] = o + b2_ref[...]


_BR_I = 1000

_inter_call = pl.pallas_call(
    _inter_body,
    grid=(_N // _BR_I,),
    in_specs=[
        pl.BlockSpec((_NC, _BR_I, _D), lambda i: (0, i, 0)),
        pl.BlockSpec((_D, _D), lambda i: (0, 0)),
        pl.BlockSpec((1, _D), lambda i: (0, 0)),
        pl.BlockSpec((_D, _D), lambda i: (0, 0)),
        pl.BlockSpec((1, _D), lambda i: (0, 0)),
    ],
    out_specs=pl.BlockSpec((_BR_I, _D), lambda i: (i, 0)),
    out_shape=jax.ShapeDtypeStruct((_N, _D), jnp.float32),
)


def kernel(x, edge_index, distances, edge_attr,
           W1f, b1f, W2f, b2f, W1i, b1i, W2i, b2i):
    del edge_attr
    row = edge_index[0].astype(jnp.int32)
    col = edge_index[1].astype(jnp.int32)
    params = jnp.stack([
        W1f[0],
        b1f,
        jnp.sum(W2f, axis=1),
        jnp.full((_F,), jnp.sum(b2f), jnp.float32),
    ])
    s = _filter_call(distances.reshape(_ROWS_S, _COLS_S), params).reshape(_E)
    packed = jnp.stack([
        row.reshape(_NCHUNKS, _CH),
        col.reshape(_NCHUNKS, _CH),
        lax.bitcast_convert_type(s.reshape(_NCHUNKS, _CH), jnp.int32),
    ], axis=1)
    partials = _sc_scatter(x, packed)
    out = _inter_call(partials.reshape(_NC, _N, _D),
                      W1i, b1i.reshape(1, _D), W2i, b2i.reshape(1, _D))
    return out
